# baseline (speedup 1.0000x reference)
import functools

import jax
import jax.numpy as jnp
from jax import lax
from jax.experimental import pallas as pl
from jax.experimental.pallas import tpu as pltpu
from jax.experimental.pallas import tpu_sc as plsc

NC = 2
NS = 16
NW = NC * NS
L = 16
GW = 128


@functools.lru_cache(maxsize=None)
def _make_gather(H, B, V, D):
    assert D == 64 and V % 2 == 0
    VP = V // 2
    bw = B // NW
    ng = bw // GW
    assert bw % GW == 0

    mesh = plsc.VectorSubcoreMesh(
        core_axis_name="c", subcore_axis_name="s",
        num_cores=NC, num_subcores=NS)

    @functools.partial(
        pl.kernel,
        out_type=jax.ShapeDtypeStruct((H, D, B), jnp.float32),
        mesh=mesh,
        compiler_params=pltpu.CompilerParams(
            use_tc_tiling_on_sc=True, needs_layout_passes=False),
        scratch_types=[
            [pltpu.VMEM((bw,), jnp.int32) for _ in range(2)],
            [pltpu.VMEM((bw,), jnp.int32) for _ in range(2)],
            [pltpu.VMEM((GW, 2 * D), jnp.float32) for _ in range(ng)],
            pltpu.VMEM((D, GW), jnp.float32),
            [pltpu.SemaphoreType.DMA for _ in range(ng)],
        ],
    )
    def gather_kernel(w2_hbm, idx_hbm, out_hbm, qs, ps, bufs, tbuf, gsems):
        wid = lax.axis_index("s") * NC + lax.axis_index("c")
        b0 = wid * bw

        def stage_indices(h, sl):
            q_v, p_v = qs[sl], ps[sl]
            pltpu.sync_copy(idx_hbm.at[h, pl.ds(b0, bw)], q_v)

            @pl.loop(0, bw // L)
            def _(i):
                r = q_v[pl.ds(i * L, L)]
                q_v[pl.ds(i * L, L)] = jnp.minimum(
                    lax.shift_right_logical(r, 1), VP - 1)
                p_v[pl.ds(i * L, L)] = lax.shift_left(
                    lax.bitwise_and(r, 1), 6)

        def fire(sl, g):
            pltpu.async_copy(
                w2_hbm.at[qs[sl].at[pl.ds(g * GW, GW)]], bufs[g], gsems[g])

        def wait(g):
            pltpu.make_async_copy(
                w2_hbm.at[pl.ds(0, GW)], bufs[g], gsems[g]).wait()

        def transpose_store(h, sl, g):
            @pl.loop(0, GW // L)
            def _(l):
                rows = lax.iota(jnp.int32, L) + l * L
                par = ps[sl][pl.ds(g * GW + l * L, L)]

                @pl.loop(0, D, unroll=8)
                def _(d):
                    tbuf[d, pl.ds(l * L, L)] = plsc.load_gather(
                        bufs[g], [rows, par + d])
            pltpu.sync_copy(tbuf, out_hbm.at[h, :, pl.ds(b0 + g * GW, GW)])

        stage_indices(0, 0)
        for g in range(ng):
            fire(0, g)

        @pl.loop(0, H, step=2)
        def _(h0):
            for r in range(2):
                h = h0 + r
                sl, nsl = r, 1 - r
                stage_indices(jnp.minimum(h + 1, H - 1), nsl)
                for g in range(ng):
                    wait(g)
                    transpose_store(h, sl, g)
                    fire(nsl, g)

        for g in range(ng):
            wait(g)

    return gather_kernel


def kernel(token_ids, weight):
    B, H = token_ids.shape
    V, D = weight.shape
    w2 = weight.reshape(V // 2, 2 * D)
    tid_t = token_ids.astype(jnp.int32).T
    out_t = _make_gather(H, B, V, D)(w2, tid_t)
    return out_t.transpose(2, 0, 1)


# speedup vs baseline: 1.5599x; 1.5599x over previous
import functools

import jax
import jax.numpy as jnp
from jax import lax
from jax.experimental import pallas as pl
from jax.experimental.pallas import tpu as pltpu
from jax.experimental.pallas import tpu_sc as plsc

NC = 2
NS = 16
NW = NC * NS
IDXW = 50
BCH = 4
NBUF = 4


@functools.lru_cache(maxsize=None)
def _make_gather(B, H, V, D):
    CH = BCH * H
    G = CH // IDXW
    b_per_w = B // NW
    t_per_w = b_per_w * H
    n_idx_rows = t_per_w // IDXW
    nch = b_per_w // BCH
    assert CH % IDXW == 0 and b_per_w % BCH == 0
    assert nch % NBUF == 0 and nch >= 2 * NBUF
    nq = nch // NBUF - 1

    mesh = plsc.VectorSubcoreMesh(
        core_axis_name="c", subcore_axis_name="s",
        num_cores=NC, num_subcores=NS)

    @functools.partial(
        pl.kernel,
        out_type=jax.ShapeDtypeStruct((B, H, D), jnp.float32),
        mesh=mesh,
        compiler_params=pltpu.CompilerParams(use_tc_tiling_on_sc=False),
        scratch_types=[
            pltpu.VMEM((n_idx_rows, IDXW), jnp.int32),
            [pltpu.VMEM((BCH, H, D), jnp.float32) for _ in range(NBUF)],
            [pltpu.SemaphoreType.DMA for _ in range(NBUF)],
        ],
    )
    def gather_kernel(table_hbm, idx_hbm, out_hbm, idx_v, bufs, gsems):
        wid = lax.axis_index("s") * NC + lax.axis_index("c")
        b_base = wid * b_per_w

        pltpu.sync_copy(idx_hbm.at[pl.ds(wid * n_idx_rows, n_idx_rows)], idx_v)

        def fire_slot(chunk, slot):
            for g in range(G):
                pltpu.async_copy(
                    table_hbm.at[idx_v.at[chunk * G + g]],
                    bufs[slot].at[g],
                    gsems[slot])

        def wait_slot(slot):
            pltpu.make_async_copy(
                out_hbm.at[pl.ds(0, BCH)], bufs[slot], gsems[slot]).wait()

        def store(chunk, slot):
            pltpu.sync_copy(
                bufs[slot],
                out_hbm.at[pl.ds(b_base + chunk * BCH, BCH)])

        for c in range(NBUF - 1):
            fire_slot(c, c)

        @pl.loop(0, nq)
        def _(q):
            i0 = q * NBUF
            for r in range(NBUF):
                wait_slot(r)
                fire_slot(i0 + r + NBUF - 1, (r + NBUF - 1) % NBUF)
                store(i0 + r, r)

        i0 = nq * NBUF
        fire_slot(i0 + NBUF - 1, NBUF - 1)
        for r in range(NBUF):
            wait_slot(r)
            store(i0 + r, r)

    return gather_kernel


def kernel(token_ids, weight):
    B, H = token_ids.shape
    V, D = weight.shape
    return _make_gather(B, H, V, D)(weight, token_ids.astype(jnp.int32))
